# Initial kernel scaffold; baseline (speedup 1.0000x reference)
import jax
import jax.numpy as jnp
from jax.experimental import pallas as pl


def kernel(xd, yd, mask):
    raise NotImplementedError("write your pallas kernel here")


# speedup vs baseline: 8.4708x; 8.4708x over previous
import jax
import jax.numpy as jnp
from jax.experimental import pallas as pl
from jax.experimental.pallas import tpu as pltpu

_N = 8192
_M = 8192
_D = 128
_ROWS = 256
_G = _N // _ROWS


def _body(xd_ref, yd_ref, mask_ref, newmask_ref, pairs_ref, topd_ref, dscr):
    i = pl.program_id(0)

    @pl.when(i == 0)
    def _():
        diff = xd_ref[0] - yd_ref[0]
        dscr[0, 0] = jnp.sqrt(jnp.sum(diff * diff))

    d = dscr[0, 0]
    dm = mask_ref[0] * d
    rmin = jnp.min(dm, axis=1, keepdims=True)
    iota = jax.lax.broadcasted_iota(jnp.int32, (_ROWS, _M), 1)
    cand = jnp.where(dm == rmin, iota, jnp.int32(_M))
    argm = jnp.min(cand, axis=1, keepdims=True)
    newmask_ref[0] = jnp.where(iota == argm, jnp.float32(1.0), jnp.float32(0.0))
    topd_ref[0, 0] = rmin[:, 0]
    pairs_ref[0, 0] = argm[:, 0]


def kernel(xd, yd, mask):
    new_mask, pairs3, topd3 = pl.pallas_call(
        _body,
        grid=(_G,),
        in_specs=[
            pl.BlockSpec((1, _N, _D), lambda i: (0, 0, 0)),
            pl.BlockSpec((1, _M, _D), lambda i: (0, 0, 0)),
            pl.BlockSpec((1, _ROWS, _M), lambda i: (0, i, 0)),
        ],
        out_specs=[
            pl.BlockSpec((1, _ROWS, _M), lambda i: (0, i, 0)),
            pl.BlockSpec((1, 1, _ROWS), lambda i: (i, 0, 0)),
            pl.BlockSpec((1, 1, _ROWS), lambda i: (i, 0, 0)),
        ],
        out_shape=[
            jax.ShapeDtypeStruct((1, _N, _M), jnp.float32),
            jax.ShapeDtypeStruct((_G, 1, _ROWS), jnp.int32),
            jax.ShapeDtypeStruct((_G, 1, _ROWS), jnp.float32),
        ],
        scratch_shapes=[pltpu.SMEM((1, 1), jnp.float32)],
    )(xd, yd, mask)
    pairs = pairs3.reshape(1, _N)
    top_dists = topd3.reshape(1, _N)
    return new_mask, pairs, top_dists
